# Initial kernel scaffold; baseline (speedup 1.0000x reference)
import jax
import jax.numpy as jnp
from jax.experimental import pallas as pl


def kernel(x, pos, edge_index, W_e1, b_e1, W_e2, b_e2, W_n1, b_n1, W_n2, b_n2, ln_g, ln_b):
    raise NotImplementedError("write your pallas kernel here")


# speedup vs baseline: 3.7053x; 3.7053x over previous
import functools

import jax
import jax.numpy as jnp
from jax import lax
from jax.experimental import pallas as pl
from jax.experimental.pallas import tpu as pltpu
from jax.experimental.pallas import tpu_sc as plsc

N = 10000
D = 128
DH = D // 2
W_H = 80
C = 128
NC, NS = 2, 16
ROWS_PER_TILE = N // NS


def _tc1_body(x_ref, wa_ref, wb_ref, be1_ref, p_ref, q_ref):
    xb = x_ref[...]
    p_ref[...] = jnp.dot(xb, wa_ref[0], preferred_element_type=jnp.float32) + be1_ref[0]
    q_ref[...] = jnp.dot(xb, wb_ref[0], preferred_element_type=jnp.float32)


def _tc1(x, w_a, w_b, b_e1):
    r = 1000
    grid = (N // r, NC)
    return pl.pallas_call(
        _tc1_body,
        grid=grid,
        in_specs=[
            pl.BlockSpec((r, D), lambda i, j: (i, 0)),
            pl.BlockSpec((1, D, DH), lambda i, j: (j, 0, 0)),
            pl.BlockSpec((1, D, DH), lambda i, j: (j, 0, 0)),
            pl.BlockSpec((1, 1, DH), lambda i, j: (j, 0, 0)),
        ],
        out_specs=[
            pl.BlockSpec((r, DH), lambda i, j: (i + (N // r) * j, 0)),
            pl.BlockSpec((r, DH), lambda i, j: (i + (N // r) * j, 0)),
        ],
        out_shape=[
            jax.ShapeDtypeStruct((NC * N, DH), jnp.float32),
            jax.ShapeDtypeStruct((NC * N, DH), jnp.float32),
        ],
    )(x, w_a, w_b, b_e1)


def _make_sc_edge(n_edges):
    mesh = plsc.VectorSubcoreMesh(core_axis_name="c", subcore_axis_name="s")
    per_sub = n_edges // NS
    n_full = per_sub // C
    tail = per_sub - n_full * C

    @functools.partial(
        pl.kernel,
        out_type=jax.ShapeDtypeStruct((NC, N, W_H), jnp.float32),
        mesh=mesh,
        compiler_params=pltpu.CompilerParams(use_tc_tiling_on_sc=False),
        scratch_types=[
            pltpu.VMEM((C,), jnp.int32),
            pltpu.VMEM((C,), jnp.int32),
            pltpu.VMEM((C,), jnp.int32),
            pltpu.VMEM((C,), jnp.int32),
            pltpu.VMEM((C,), jnp.int32),
            pltpu.VMEM((C,), jnp.int32),
            pltpu.VMEM((C,), jnp.int32),
            pltpu.VMEM((C,), jnp.int32),
            pltpu.VMEM((C,), jnp.int32),
            pltpu.VMEM((C,), jnp.int32),
            pltpu.VMEM((C, DH), jnp.float32),
            pltpu.VMEM((C, DH), jnp.float32),
            pltpu.VMEM((C,), jnp.float32),
            pltpu.VMEM((C,), jnp.float32),
            pltpu.VMEM((C,), jnp.float32),
            pltpu.VMEM((C,), jnp.float32),
            pltpu.VMEM((C,), jnp.float32),
            pltpu.VMEM((C,), jnp.float32),
            pltpu.VMEM((C, W_H), jnp.float32),
            pltpu.VMEM((32,), jnp.int32),
            pltpu.VMEM((DH,), jnp.float32),
            pltpu.VMEM_SHARED((N, W_H), jnp.float32),
            pltpu.SemaphoreType.DMA,
        ],
    )
    def sc_edge(p_hbm, q_hbm, pos_hbm, ei_hbm, wd_hbm, out_hbm,
                rowix, colix, rofs, cofs, ri0, ri1, ri2, ci0, ci1, ci2,
                pi, qj, pxi, pyi, pzi, pxj, pyj, pzj, hbuf, rowix_t, wd, hagg, sem):
        cid = lax.axis_index("c")
        sid = lax.axis_index("s")
        r0 = sid * ROWS_PER_TILE

        zv = jnp.zeros((16,), jnp.float32)

        def zero_hbuf(e, carry):
            for j in range(W_H // 16):
                hbuf[e, pl.ds(16 * j, 16)] = zv
            return carry

        lax.fori_loop(0, C, zero_hbuf, 0)

        def zero_acc(t, carry):
            pltpu.sync_copy(hbuf.at[pl.ds(0, 125)],
                            hagg.at[pl.ds(r0 + t * 125, 125)])
            return carry

        lax.fori_loop(0, ROWS_PER_TILE // 125, zero_acc, 0)

        pltpu.sync_copy(wd_hbm.at[cid], wd)

        ones0 = jnp.where(lax.iota(jnp.int32, 16) == 0,
                          jnp.full((16,), 1.0, jnp.float32), zv)

        def set_deg_col(e, carry):
            hbuf[e, pl.ds(DH, 16)] = ones0
            return carry

        lax.fori_loop(0, C, set_deg_col, 0)
        plsc.subcore_barrier()

        base = sid * per_sub
        tb = cid * N
        wds = [wd[pl.ds(16 * j, 16)] for j in range(DH // 16)]
        one_i = jnp.full((16,), 1, jnp.int32)
        two_i = jnp.full((16,), 2, jnp.int32)

        def do_chunk(e0, cc):
            pltpu.sync_copy(ei_hbm.at[pl.ds(e0, cc)], rowix.at[pl.ds(0, cc)])
            pltpu.sync_copy(ei_hbm.at[pl.ds(n_edges + e0, cc)],
                            colix.at[pl.ds(0, cc)])
            if cc == C:
                scat = rowix
            else:
                scat = rowix_t
                pltpu.sync_copy(ei_hbm.at[pl.ds(e0, cc)], scat)

            def idx_body(g, icarry):
                sl = pl.ds(g * 16, 16)
                rv = rowix[sl]
                cv = colix[sl]
                rofs[sl] = rv + tb
                cofs[sl] = cv + tb
                r3 = rv + rv + rv
                c3 = cv + cv + cv
                ri0[sl] = r3
                ri1[sl] = r3 + one_i
                ri2[sl] = r3 + two_i
                ci0[sl] = c3
                ci1[sl] = c3 + one_i
                ci2[sl] = c3 + two_i
                return icarry

            lax.fori_loop(0, cc // 16, idx_body, 0)
            cps = [
                pltpu.async_copy(p_hbm.at[rofs.at[pl.ds(0, cc)]],
                                 pi.at[pl.ds(0, cc)], sem),
                pltpu.async_copy(q_hbm.at[cofs.at[pl.ds(0, cc)]],
                                 qj.at[pl.ds(0, cc)], sem),
                pltpu.async_copy(pos_hbm.at[ri0.at[pl.ds(0, cc)]],
                                 pxi.at[pl.ds(0, cc)], sem),
                pltpu.async_copy(pos_hbm.at[ri1.at[pl.ds(0, cc)]],
                                 pyi.at[pl.ds(0, cc)], sem),
                pltpu.async_copy(pos_hbm.at[ri2.at[pl.ds(0, cc)]],
                                 pzi.at[pl.ds(0, cc)], sem),
                pltpu.async_copy(pos_hbm.at[ci0.at[pl.ds(0, cc)]],
                                 pxj.at[pl.ds(0, cc)], sem),
                pltpu.async_copy(pos_hbm.at[ci1.at[pl.ds(0, cc)]],
                                 pyj.at[pl.ds(0, cc)], sem),
                pltpu.async_copy(pos_hbm.at[ci2.at[pl.ds(0, cc)]],
                                 pzj.at[pl.ds(0, cc)], sem),
            ]
            for cp in cps:
                cp.wait()

            def h_body(g, hcarry):
                sl16 = pl.ds(g * 16, 16)
                dx = pxi[sl16] - pxj[sl16]
                dy = pyi[sl16] - pyj[sl16]
                dz = pzi[sl16] - pzj[sl16]
                d2 = dx * dx + dy * dy + dz * dz
                bits = lax.bitcast_convert_type(d2, jnp.int32)
                y = lax.bitcast_convert_type(
                    jnp.full((16,), 0x5F3759DF, jnp.int32) - (bits >> 1),
                    jnp.float32)
                half = d2 * 0.5
                y = y * (1.5 - half * y * y)
                y = y * (1.5 - half * y * y)
                y = y * (1.5 - half * y * y)
                dv = jnp.where(d2 > 0.0, d2 * y, zv)
                for l in range(16):
                    ds = dv[l]
                    e = g * 16 + l
                    for j in range(DH // 16):
                        sl = pl.ds(16 * j, 16)
                        hbuf[e, sl] = jnp.maximum(pi[e, sl] + qj[e, sl] + ds * wds[j], 0.0)
                return hcarry

            lax.fori_loop(0, cc // 16, h_body, 0)
            pltpu.sync_copy(hbuf.at[pl.ds(0, cc)], hagg.at[scat], add=True)

        def chunk(k, carry):
            do_chunk(base + k * C, C)
            return carry

        lax.fori_loop(0, n_full, chunk, 0)
        if tail:
            do_chunk(base + n_full * C, tail)

        plsc.subcore_barrier()
        pltpu.sync_copy(hagg.at[pl.ds(r0, ROWS_PER_TILE)],
                        out_hbm.at[cid, pl.ds(r0, ROWS_PER_TILE)])

    return sc_edge


def _tc2_body(ha_ref, x_ref, we2a_ref, we2b_ref, be2_ref, wn1_ref, bn1_ref,
              wn2_ref, bn2_ref, g_ref, b_ref, o_ref):
    a = ha_ref[...]
    h0 = a[0, :, 0:DH]
    h1 = a[1, :, 0:DH]
    deg = a[0, :, DH:DH + 1]
    agg = (jnp.dot(h0, we2a_ref[...], preferred_element_type=jnp.float32)
           + jnp.dot(h1, we2b_ref[...], preferred_element_type=jnp.float32)
           + deg * be2_ref[...])
    xb = x_ref[...]
    w1 = wn1_ref[...]
    h2 = jnp.maximum(
        jnp.dot(xb, w1[0:D], preferred_element_type=jnp.float32)
        + jnp.dot(agg, w1[D:2 * D], preferred_element_type=jnp.float32)
        + bn1_ref[...], 0.0)
    out = jnp.dot(h2, wn2_ref[...], preferred_element_type=jnp.float32) + bn2_ref[...] + xb
    mu = jnp.mean(out, axis=-1, keepdims=True)
    cen = out - mu
    var = jnp.mean(cen * cen, axis=-1, keepdims=True)
    o_ref[...] = cen * lax.rsqrt(var + 1e-5) * g_ref[...] + b_ref[...]


def _tc2(hagg, x, w_e2, b_e2, w_n1, b_n1, w_n2, b_n2, ln_g, ln_b):
    r = 1000
    grid = (N // r,)
    return pl.pallas_call(
        _tc2_body,
        grid=grid,
        in_specs=[
            pl.BlockSpec((NC, r, W_H), lambda i: (0, i, 0)),
            pl.BlockSpec((r, D), lambda i: (i, 0)),
            pl.BlockSpec((DH, D), lambda i: (0, 0)),
            pl.BlockSpec((DH, D), lambda i: (1, 0)),
            pl.BlockSpec((1, D), lambda i: (0, 0)),
            pl.BlockSpec((2 * D, D), lambda i: (0, 0)),
            pl.BlockSpec((1, D), lambda i: (0, 0)),
            pl.BlockSpec((D, D), lambda i: (0, 0)),
            pl.BlockSpec((1, D), lambda i: (0, 0)),
            pl.BlockSpec((1, D), lambda i: (0, 0)),
            pl.BlockSpec((1, D), lambda i: (0, 0)),
        ],
        out_specs=pl.BlockSpec((r, D), lambda i: (i, 0)),
        out_shape=jax.ShapeDtypeStruct((N, D), jnp.float32),
    )(hagg, x, w_e2, w_e2, b_e2, w_n1, b_n1, w_n2, b_n2, ln_g, ln_b)


def kernel(x, pos, edge_index, W_e1, b_e1, W_e2, b_e2, W_n1, b_n1, W_n2, b_n2,
           ln_g, ln_b):
    n_edges = edge_index.shape[1]

    w_a = jnp.stack([W_e1[:D, :DH], W_e1[:D, DH:]])
    w_b = jnp.stack([W_e1[D:2 * D, :DH], W_e1[D:2 * D, DH:]])
    wd2 = W_e1[2 * D].reshape(NC, DH)
    pos_flat = pos.reshape(-1)
    ei_flat = edge_index.reshape(-1)

    p2, q2 = _tc1(x, w_a, w_b, b_e1.reshape(NC, 1, DH))
    hagg = _make_sc_edge(n_edges)(p2, q2, pos_flat, ei_flat, wd2)
    out = _tc2(hagg, x, W_e2, b_e2.reshape(1, D), W_n1, b_n1.reshape(1, D),
               W_n2, b_n2.reshape(1, D), ln_g.reshape(1, D), ln_b.reshape(1, D))
    return (out, pos)
